# Initial kernel scaffold; baseline (speedup 1.0000x reference)
import jax
import jax.numpy as jnp
from jax.experimental import pallas as pl


def kernel(x, kernels, one_padding):
    raise NotImplementedError("write your pallas kernel here")


# speedup vs baseline: 5.2004x; 5.2004x over previous
import functools

import jax
import jax.numpy as jnp
from jax import lax
from jax.experimental import pallas as pl
from jax.experimental.pallas import tpu as pltpu
from jax.experimental.pallas import tpu_sc as plsc

_K = 20
_KS = 9
_ROWS = 256


def _knn_body(xt_ref, xb_ref, idx_ref, dist_ref):
    b = pl.program_id(0)
    n = xb_ref.shape[2]
    rows = xt_ref.shape[1]
    a = xt_ref[0]
    xb = xb_ref[0]
    xxr = jnp.sum(a * a, axis=1, keepdims=True)
    xxc = jnp.sum(xb * xb, axis=0, keepdims=True)
    inner = -2.0 * jnp.dot(a.astype(jnp.bfloat16), xb.astype(jnp.bfloat16),
                           preferred_element_type=jnp.float32)
    dist = ((-xxc) - inner) - xxr
    u = lax.bitcast_convert_type(dist, jnp.int32)
    dist_ref[...] = jnp.where(u < 0, jnp.int32(0x7FFFFFFF) ^ u, u)
    colio = lax.broadcasted_iota(jnp.int32, (rows, n), 1)
    base = b * n
    for t in range(_K):
        v = dist_ref[...]
        m = jnp.max(v, axis=1, keepdims=True)
        eq = v == m
        win = jnp.min(jnp.where(eq, colio, n), axis=1)
        idx_ref[0, t, :] = win + base
        dist_ref[...] = jnp.where(eq & (colio == win[:, None]),
                                  jnp.iinfo(jnp.int32).min, v)


def _knn_topk(x, xt):
    bsize, _, n = x.shape
    grid = (bsize, n // _ROWS)
    out = pl.pallas_call(
        _knn_body,
        grid=grid,
        in_specs=[
            pl.BlockSpec((1, _ROWS, 3), lambda b, r: (b, r, 0)),
            pl.BlockSpec((1, 3, n), lambda b, r: (b, 0, 0)),
        ],
        out_specs=pl.BlockSpec((1, 32, _ROWS), lambda b, r: (b, 0, r)),
        out_shape=jax.ShapeDtypeStruct((bsize, 32, n), jnp.int32),
        scratch_shapes=[pltpu.VMEM((_ROWS, n), jnp.int32)],
    )(xt, x)
    return out


def _bf16_round(v):
    u = lax.bitcast_convert_type(v, jnp.uint32)
    lsb = (u >> 16) & jnp.uint32(1)
    u = (u + jnp.uint32(0x7FFF) + lsb) & jnp.uint32(0xFFFF0000)
    return lax.bitcast_convert_type(u, jnp.float32)


_CH = 16
_NIDX = _CH * _K
_NEL = 3 * _NIDX
_SCH = 128


def _sc_body(x3_hbm, sidx_hbm, kf_hbm, opf_hbm,
             out_hbm, idx_v, iel_v, xel_v, pm_v, kvf, opvf, sem):
    nw = 32
    npts = x3_hbm.shape[0] // 3
    per_tile = npts // nw
    nsch = per_tile // _SCH
    wid = lax.axis_index("s") * 2 + lax.axis_index("c")
    tile_base = wid * per_tile

    pltpu.sync_copy(kf_hbm, kvf)
    pltpu.sync_copy(opf_hbm, opvf)

    iota16 = lax.iota(jnp.int32, 16)
    rb20 = iota16 * _K

    krows = [kvf[pl.ds(f * 16, 16)] for f in range(3)]
    kc = [[krows[f][s] for s in range(_KS)] for f in range(3)]
    oprow = opvf[pl.ds(0, 16)]
    pm0 = [jnp.maximum(jnp.full((16,), oprow[s]), 0.0) for s in range(_KS)]

    def sub_body(j, col0):
        cb = j * _CH
        ib = (col0 + cb) * _K
        pltpu.sync_copy(sidx_hbm.at[pl.ds(ib, _NIDX)], idx_v)
        for g in range(_NIDX // 16):
            v = idx_v[pl.ds(g * 16, 16)]
            iel_v[pl.ds(g * 16, 16)] = v
            iel_v[pl.ds(_NIDX + g * 16, 16)] = v + npts
            iel_v[pl.ds(2 * _NIDX + g * 16, 16)] = v + 2 * npts
        cps = []
        for o in range(0, _NEL, 128):
            sz = min(128, _NEL - o)
            cps.append(pltpu.async_copy(
                x3_hbm.at[iel_v.at[pl.ds(o, sz)]],
                xel_v.at[pl.ds(o, sz)], sem))
        for cp in cps:
            cp.wait()

        cf = [plsc.load_gather(xel_v, [rb20 + f * _NIDX]) for f in range(3)]

        for s in range(_KS):
            pm_v[s, pl.ds(cb, _CH)] = pm0[s]

        def k_body(k, s1):
            rel = [_bf16_round(
                plsc.load_gather(xel_v, [rb20 + (k + f * _NIDX)]) - cf[f])
                   for f in range(3)]
            s1n = []
            for s in range(_KS):
                pm = rel[0] * kc[0][s] + rel[1] * kc[1][s] + rel[2] * kc[2][s]
                pm = jnp.maximum(pm, 0.0)
                s1n.append(s1[s] + pm)
                pm_v[k * _KS + s, pl.ds(cb, _CH)] = pm
            return tuple(s1n)

        s1 = lax.fori_loop(1, _K, k_body, tuple(pm0))
        r1 = [1.0 / (s1[s] + 1e-6) for s in range(_KS)]

        def k2_body(k, s2):
            s2n = []
            for s in range(_KS):
                p1 = pm_v[k * _KS + s, pl.ds(cb, _CH)] * r1[s]
                p2 = p1 * p1
                s2n.append(s2[s] + p2)
                pm_v[k * _KS + s, pl.ds(cb, _CH)] = p2
            return tuple(s2n)

        s2 = lax.fori_loop(0, _K, k2_body, tuple(
            jnp.zeros((16,), jnp.float32) for _ in range(_KS)))
        r2 = [1.0 / (s2[s] + 1e-6) for s in range(_KS)]

        def k3_body(k, c):
            for s in range(_KS):
                p3 = pm_v[k * _KS + s, pl.ds(cb, _CH)] * r2[s]
                p3 = jnp.where(p3 > 0.1, p3, 0.0)
                pm_v[k * _KS + s, pl.ds(cb, _CH)] = p3
            return c

        lax.fori_loop(0, _K, k3_body, 0)
        return col0

    def sch_body(sc, carry):
        col0 = tile_base + sc * _SCH
        lax.fori_loop(0, _SCH // _CH, sub_body, col0)
        pltpu.sync_copy(pm_v, out_hbm.at[:, pl.ds(col0, _SCH)])
        return carry

    lax.fori_loop(0, nsch, sch_body, 0)


def _sc_permatrix(x3, sidx, kf, opf):
    npts = x3.shape[0] // 3
    mesh = plsc.VectorSubcoreMesh(core_axis_name="c", subcore_axis_name="s")
    f = functools.partial(
        pl.kernel,
        mesh=mesh,
        compiler_params=pltpu.CompilerParams(needs_layout_passes=False),
        out_type=jax.ShapeDtypeStruct((_K * _KS, npts), jnp.float32),
        scratch_types=[
            pltpu.VMEM((_NIDX,), jnp.int32),
            pltpu.VMEM((_NEL,), jnp.int32),
            pltpu.VMEM((_NEL,), jnp.float32),
            pltpu.VMEM((_K * _KS, _SCH), jnp.float32),
            pltpu.VMEM((48,), jnp.float32),
            pltpu.VMEM((320,), jnp.float32),
            pltpu.SemaphoreType.DMA,
        ],
    )(_sc_body)
    return f(x3, sidx, kf, opf)


def kernel(x, kernels, one_padding):
    bsize, feats, n = x.shape
    xt = jnp.transpose(x, (0, 2, 1))
    idx_out = _knn_topk(x, xt)
    sidx = jnp.transpose(idx_out[:, :_K, :], (0, 2, 1)).reshape(-1)

    x3 = jnp.transpose(x, (1, 0, 2)).reshape(-1)
    kbf = kernels.astype(jnp.bfloat16).astype(jnp.float32)
    kf = jnp.pad(kbf, ((0, 0), (0, 16 - _KS))).reshape(-1)
    opf = jnp.pad(one_padding, ((0, 0), (0, 16 - _KS))).reshape(-1)
    pm = _sc_permatrix(x3, sidx, kf, opf)
    permatrix = pm.T.reshape(bsize * n, _K, _KS)
    return (sidx, permatrix)


# speedup vs baseline: 7.7333x; 1.4871x over previous
import functools

import jax
import jax.numpy as jnp
from jax import lax
from jax.experimental import pallas as pl
from jax.experimental.pallas import tpu as pltpu
from jax.experimental.pallas import tpu_sc as plsc

_K = 20
_KS = 9
_ROWS = 256


def _knn_body(xt_ref, xb_ref, idx_ref, dist_ref):
    b = pl.program_id(0)
    n = xb_ref.shape[2]
    rows = xt_ref.shape[1]
    a = xt_ref[0]
    xb = xb_ref[0]
    xxr = jnp.sum(a * a, axis=1, keepdims=True)
    xxc = jnp.sum(xb * xb, axis=0, keepdims=True)
    inner = -2.0 * jnp.dot(a.astype(jnp.bfloat16), xb.astype(jnp.bfloat16),
                           preferred_element_type=jnp.float32)
    dist_ref[...] = ((-xxc) - inner) - xxr
    colio = lax.broadcasted_iota(jnp.int32, (rows, n), 1)
    base = b * n
    for t in range(_K):
        v = dist_ref[...]
        win = jnp.argmax(v, axis=1).astype(jnp.int32)
        idx_ref[0, t, :] = win + base
        dist_ref[...] = jnp.where(colio == win[:, None], -jnp.inf, v)


def _knn_topk(x, xt):
    bsize, _, n = x.shape
    grid = (bsize, n // _ROWS)
    out = pl.pallas_call(
        _knn_body,
        grid=grid,
        in_specs=[
            pl.BlockSpec((1, _ROWS, 3), lambda b, r: (b, r, 0)),
            pl.BlockSpec((1, 3, n), lambda b, r: (b, 0, 0)),
        ],
        out_specs=pl.BlockSpec((1, 32, _ROWS), lambda b, r: (b, 0, r)),
        out_shape=jax.ShapeDtypeStruct((bsize, 32, n), jnp.int32),
        scratch_shapes=[pltpu.VMEM((_ROWS, n), jnp.float32)],
    )(xt, x)
    return out


def _bf16_round(v):
    u = lax.bitcast_convert_type(v, jnp.uint32)
    lsb = (u >> 16) & jnp.uint32(1)
    u = (u + jnp.uint32(0x7FFF) + lsb) & jnp.uint32(0xFFFF0000)
    return lax.bitcast_convert_type(u, jnp.float32)


_CH = 16
_NIDX = _CH * _K
_NEL = 3 * _NIDX
_SCH = 128


def _sc_body(x3_hbm, sidx_hbm, kf_hbm, opf_hbm,
             out_hbm, idx_v, iel_v, xel_v, pm_v, kvf, opvf, sem):
    nw = 32
    npts = x3_hbm.shape[0] // 3
    per_tile = npts // nw
    nsch = per_tile // _SCH
    wid = lax.axis_index("s") * 2 + lax.axis_index("c")
    tile_base = wid * per_tile

    pltpu.sync_copy(kf_hbm, kvf)
    pltpu.sync_copy(opf_hbm, opvf)

    iota16 = lax.iota(jnp.int32, 16)
    rb20 = iota16 * _K

    krows = [kvf[pl.ds(f * 16, 16)] for f in range(3)]
    kc = [[krows[f][s] for s in range(_KS)] for f in range(3)]
    oprow = opvf[pl.ds(0, 16)]
    pm0 = [jnp.maximum(jnp.full((16,), oprow[s]), 0.0) for s in range(_KS)]

    def sub_body(j, col0):
        cb = j * _CH
        ib = (col0 + cb) * _K
        pltpu.sync_copy(sidx_hbm.at[pl.ds(ib, _NIDX)], idx_v)
        for g in range(_NIDX // 16):
            v = idx_v[pl.ds(g * 16, 16)]
            iel_v[pl.ds(g * 16, 16)] = v
            iel_v[pl.ds(_NIDX + g * 16, 16)] = v + npts
            iel_v[pl.ds(2 * _NIDX + g * 16, 16)] = v + 2 * npts
        cps = []
        for o in range(0, _NEL, 128):
            sz = min(128, _NEL - o)
            cps.append(pltpu.async_copy(
                x3_hbm.at[iel_v.at[pl.ds(o, sz)]],
                xel_v.at[pl.ds(o, sz)], sem))
        for cp in cps:
            cp.wait()

        cf = [plsc.load_gather(xel_v, [rb20 + f * _NIDX]) for f in range(3)]

        for s in range(_KS):
            pm_v[s, pl.ds(cb, _CH)] = pm0[s]

        def k_body(k, s1):
            rel = [_bf16_round(
                plsc.load_gather(xel_v, [rb20 + (k + f * _NIDX)]) - cf[f])
                   for f in range(3)]
            s1n = []
            for s in range(_KS):
                pm = rel[0] * kc[0][s] + rel[1] * kc[1][s] + rel[2] * kc[2][s]
                pm = jnp.maximum(pm, 0.0)
                s1n.append(s1[s] + pm)
                pm_v[k * _KS + s, pl.ds(cb, _CH)] = pm
            return tuple(s1n)

        s1 = lax.fori_loop(1, _K, k_body, tuple(pm0))
        r1 = [1.0 / (s1[s] + 1e-6) for s in range(_KS)]

        def k2_body(k, s2):
            s2n = []
            for s in range(_KS):
                p1 = pm_v[k * _KS + s, pl.ds(cb, _CH)] * r1[s]
                p2 = p1 * p1
                s2n.append(s2[s] + p2)
                pm_v[k * _KS + s, pl.ds(cb, _CH)] = p2
            return tuple(s2n)

        s2 = lax.fori_loop(0, _K, k2_body, tuple(
            jnp.zeros((16,), jnp.float32) for _ in range(_KS)))
        r2 = [1.0 / (s2[s] + 1e-6) for s in range(_KS)]

        def k3_body(k, c):
            for s in range(_KS):
                p3 = pm_v[k * _KS + s, pl.ds(cb, _CH)] * r2[s]
                p3 = jnp.where(p3 > 0.1, p3, 0.0)
                pm_v[k * _KS + s, pl.ds(cb, _CH)] = p3
            return c

        lax.fori_loop(0, _K, k3_body, 0)
        return col0

    def sch_body(sc, carry):
        col0 = tile_base + sc * _SCH
        lax.fori_loop(0, _SCH // _CH, sub_body, col0)
        pltpu.sync_copy(pm_v, out_hbm.at[:, pl.ds(col0, _SCH)])
        return carry

    lax.fori_loop(0, nsch, sch_body, 0)


def _sc_permatrix(x3, sidx, kf, opf):
    npts = x3.shape[0] // 3
    mesh = plsc.VectorSubcoreMesh(core_axis_name="c", subcore_axis_name="s")
    f = functools.partial(
        pl.kernel,
        mesh=mesh,
        compiler_params=pltpu.CompilerParams(needs_layout_passes=False),
        out_type=jax.ShapeDtypeStruct((_K * _KS, npts), jnp.float32),
        scratch_types=[
            pltpu.VMEM((_NIDX,), jnp.int32),
            pltpu.VMEM((_NEL,), jnp.int32),
            pltpu.VMEM((_NEL,), jnp.float32),
            pltpu.VMEM((_K * _KS, _SCH), jnp.float32),
            pltpu.VMEM((48,), jnp.float32),
            pltpu.VMEM((320,), jnp.float32),
            pltpu.SemaphoreType.DMA,
        ],
    )(_sc_body)
    return f(x3, sidx, kf, opf)


def kernel(x, kernels, one_padding):
    bsize, feats, n = x.shape
    xt = jnp.transpose(x, (0, 2, 1))
    idx_out = _knn_topk(x, xt)
    sidx = jnp.transpose(idx_out[:, :_K, :], (0, 2, 1)).reshape(-1)

    x3 = jnp.transpose(x, (1, 0, 2)).reshape(-1)
    kbf = kernels.astype(jnp.bfloat16).astype(jnp.float32)
    kf = jnp.pad(kbf, ((0, 0), (0, 16 - _KS))).reshape(-1)
    opf = jnp.pad(one_padding, ((0, 0), (0, 16 - _KS))).reshape(-1)
    pm = _sc_permatrix(x3, sidx, kf, opf)
    permatrix = pm.T.reshape(bsize * n, _K, _KS)
    return (sidx, permatrix)


# speedup vs baseline: 8.6764x; 1.1219x over previous
import functools

import jax
import jax.numpy as jnp
from jax import lax
from jax.experimental import pallas as pl
from jax.experimental.pallas import tpu as pltpu
from jax.experimental.pallas import tpu_sc as plsc

_K = 20
_KS = 9
_ROWS = 256


def _knn_body(xt_ref, xb_ref, idx_ref, dist_ref):
    b = pl.program_id(0)
    n = xb_ref.shape[2]
    rows = xt_ref.shape[1]
    a = xt_ref[0]
    xb = xb_ref[0]
    xxr = jnp.sum(a * a, axis=1, keepdims=True)
    xxc = jnp.sum(xb * xb, axis=0, keepdims=True)
    inner = -2.0 * jnp.dot(a.astype(jnp.bfloat16), xb.astype(jnp.bfloat16),
                           preferred_element_type=jnp.float32)
    dist_ref[...] = ((-xxc) - inner) - xxr
    colio = lax.broadcasted_iota(jnp.int32, (rows, n), 1)
    base = b * n
    for t in range(_K):
        v = dist_ref[...]
        win = jnp.argmax(v, axis=1).astype(jnp.int32)
        idx_ref[0, t, :] = win + base
        dist_ref[...] = jnp.where(colio == win[:, None], -jnp.inf, v)


def _knn_topk(x, xt):
    bsize, _, n = x.shape
    grid = (bsize, n // _ROWS)
    out = pl.pallas_call(
        _knn_body,
        grid=grid,
        in_specs=[
            pl.BlockSpec((1, _ROWS, 3), lambda b, r: (b, r, 0)),
            pl.BlockSpec((1, 3, n), lambda b, r: (b, 0, 0)),
        ],
        out_specs=pl.BlockSpec((1, 32, _ROWS), lambda b, r: (b, 0, r)),
        out_shape=jax.ShapeDtypeStruct((bsize, 32, n), jnp.int32),
        scratch_shapes=[pltpu.VMEM((_ROWS, n), jnp.float32)],
    )(xt, x)
    return out


def _bf16_round(v):
    u = lax.bitcast_convert_type(v, jnp.uint32)
    lsb = (u >> 16) & jnp.uint32(1)
    u = (u + jnp.uint32(0x7FFF) + lsb) & jnp.uint32(0xFFFF0000)
    return lax.bitcast_convert_type(u, jnp.float32)


_CH = 16
_NIDX = _CH * _K
_NEL = 3 * _NIDX
_SCH = 128


def _sc_body(x3_hbm, sidx_hbm, kf_hbm, opf_hbm,
             out_hbm, idx_v, iel_v, xel_v, idx_v2, iel_v2, xel_v2,
             pm_v, kvf, opvf, sem, sem2):
    nw = 32
    npts = x3_hbm.shape[0] // 3
    per_tile = npts // nw
    nsch = per_tile // _SCH
    wid = lax.axis_index("s") * 2 + lax.axis_index("c")
    tile_base = wid * per_tile

    pltpu.sync_copy(kf_hbm, kvf)
    pltpu.sync_copy(opf_hbm, opvf)

    iota16 = lax.iota(jnp.int32, 16)
    rb20 = iota16 * _K

    krows = [kvf[pl.ds(f * 16, 16)] for f in range(3)]
    kc = [[krows[f][s] for s in range(_KS)] for f in range(3)]
    oprow = opvf[pl.ds(0, 16)]
    pm0 = [jnp.maximum(jnp.full((16,), oprow[s]), 0.0) for s in range(_KS)]

    def stage(j, col0, idxv, ielv, xelv, sm):
        ib = (col0 + j * _CH) * _K
        pltpu.sync_copy(sidx_hbm.at[pl.ds(ib, _NIDX)], idxv)
        for g in range(_NIDX // 16):
            v = idxv[pl.ds(g * 16, 16)]
            ielv[pl.ds(g * 16, 16)] = v
            ielv[pl.ds(_NIDX + g * 16, 16)] = v + npts
            ielv[pl.ds(2 * _NIDX + g * 16, 16)] = v + 2 * npts
        cps = []
        for o in range(0, _NEL, 128):
            sz = min(128, _NEL - o)
            cps.append(pltpu.async_copy(
                x3_hbm.at[ielv.at[pl.ds(o, sz)]],
                xelv.at[pl.ds(o, sz)], sm))
        return cps

    def compute(j, xel_v, cps):
        cb = j * _CH
        for cp in cps:
            cp.wait()

        cf = [plsc.load_gather(xel_v, [rb20 + f * _NIDX]) for f in range(3)]

        for s in range(_KS):
            pm_v[s, pl.ds(cb, _CH)] = pm0[s]

        def k_body(k, s1):
            rel = [_bf16_round(
                plsc.load_gather(xel_v, [rb20 + (k + f * _NIDX)]) - cf[f])
                   for f in range(3)]
            s1n = []
            for s in range(_KS):
                pm = rel[0] * kc[0][s] + rel[1] * kc[1][s] + rel[2] * kc[2][s]
                pm = jnp.maximum(pm, 0.0)
                s1n.append(s1[s] + pm)
                pm_v[k * _KS + s, pl.ds(cb, _CH)] = pm
            return tuple(s1n)

        s1 = lax.fori_loop(1, _K, k_body, tuple(pm0))
        r1 = [1.0 / (s1[s] + 1e-6) for s in range(_KS)]

        def k2_body(k, s2):
            s2n = []
            for s in range(_KS):
                p1 = pm_v[k * _KS + s, pl.ds(cb, _CH)] * r1[s]
                p2 = p1 * p1
                s2n.append(s2[s] + p2)
                pm_v[k * _KS + s, pl.ds(cb, _CH)] = p2
            return tuple(s2n)

        s2 = lax.fori_loop(0, _K, k2_body, tuple(
            jnp.zeros((16,), jnp.float32) for _ in range(_KS)))
        r2 = [1.0 / (s2[s] + 1e-6) for s in range(_KS)]

        def k3_body(k, c):
            for s in range(_KS):
                p3 = pm_v[k * _KS + s, pl.ds(cb, _CH)] * r2[s]
                p3 = jnp.where(p3 > 0.1, p3, 0.0)
                pm_v[k * _KS + s, pl.ds(cb, _CH)] = p3
            return c

        lax.fori_loop(0, _K, k3_body, 0)

    bufs = [(idx_v, iel_v, xel_v, sem), (idx_v2, iel_v2, xel_v2, sem2)]

    def sch_body(sc, carry):
        col0 = tile_base + sc * _SCH
        cps = stage(0, col0, *bufs[0])
        for j in range(_SCH // _CH):
            if j + 1 < _SCH // _CH:
                cps_next = stage(j + 1, col0, *bufs[(j + 1) % 2])
            compute(j, bufs[j % 2][2], cps)
            cps = cps_next
        pltpu.sync_copy(pm_v, out_hbm.at[:, pl.ds(col0, _SCH)])
        return carry

    lax.fori_loop(0, nsch, sch_body, 0)


def _sc_permatrix(x3, sidx, kf, opf):
    npts = x3.shape[0] // 3
    mesh = plsc.VectorSubcoreMesh(core_axis_name="c", subcore_axis_name="s")
    f = functools.partial(
        pl.kernel,
        mesh=mesh,
        compiler_params=pltpu.CompilerParams(needs_layout_passes=False),
        out_type=jax.ShapeDtypeStruct((_K * _KS, npts), jnp.float32),
        scratch_types=[
            pltpu.VMEM((_NIDX,), jnp.int32),
            pltpu.VMEM((_NEL,), jnp.int32),
            pltpu.VMEM((_NEL,), jnp.float32),
            pltpu.VMEM((_NIDX,), jnp.int32),
            pltpu.VMEM((_NEL,), jnp.int32),
            pltpu.VMEM((_NEL,), jnp.float32),
            pltpu.VMEM((_K * _KS, _SCH), jnp.float32),
            pltpu.VMEM((48,), jnp.float32),
            pltpu.VMEM((320,), jnp.float32),
            pltpu.SemaphoreType.DMA,
            pltpu.SemaphoreType.DMA,
        ],
    )(_sc_body)
    return f(x3, sidx, kf, opf)


def kernel(x, kernels, one_padding):
    bsize, feats, n = x.shape
    xt = jnp.transpose(x, (0, 2, 1))
    idx_out = _knn_topk(x, xt)
    sidx = jnp.transpose(idx_out[:, :_K, :], (0, 2, 1)).reshape(-1)

    x3 = jnp.transpose(x, (1, 0, 2)).reshape(-1)
    kbf = kernels.astype(jnp.bfloat16).astype(jnp.float32)
    kf = jnp.pad(kbf, ((0, 0), (0, 16 - _KS))).reshape(-1)
    opf = jnp.pad(one_padding, ((0, 0), (0, 16 - _KS))).reshape(-1)
    pm = _sc_permatrix(x3, sidx, kf, opf)
    permatrix = pm.T.reshape(bsize * n, _K, _KS)
    return (sidx, permatrix)
